# baseline (speedup 1.0000x reference)
import jax
import jax.numpy as jnp
from jax import lax
from jax.experimental import pallas as pl
from jax.experimental.pallas import tpu as pltpu
from jax.experimental.pallas import tpu_sc as plsc

_N = 20000
_K = 4000
_U = 8
_CH = 50
_TAB = 100064


def _greedy_body(packed_hbm, ztab_hbm, fill_hbm, kvec_hbm, out_hbm,
                 packed_v, table_v, top_v, kv):
    cid = lax.axis_index("c")
    sid = lax.axis_index("s")

    @pl.when(jnp.logical_and(cid == 0, sid == 0))
    def _():
        pltpu.sync_copy(packed_hbm, packed_v)
        pltpu.sync_copy(ztab_hbm, table_v)
        pltpu.sync_copy(fill_hbm, top_v)
        pltpu.sync_copy(kvec_hbm, kv)
        kk = kv[...][0]
        lanes = lax.iota(jnp.int32, 16)
        d1 = lanes + 16

        def crosscheck(v0, v1, w1, lim):
            bad0 = ((lanes >= 1) & (lanes <= w1) & ((v0 >> 5) > lim - lanes)) | \
                   ((lanes < w1) & ((v0 & 31) > lanes + 1))
            bad1 = ((d1 <= w1) & ((v1 >> 5) > lim - d1)) | \
                   ((d1 < w1) & ((v1 & 31) > d1 + 1))
            return jnp.any(bad0 | bad1)

        def step(i, cnt):
            t0 = _U * i
            s, w1, lim, e, v0, v1 = [], [], [], [], [], []
            for j in range(_U):
                swj = plsc.load_gather(
                    packed_v, [jnp.full((16,), t0 + j, jnp.int32)])[0]
                sj = swj >> 5
                w1j = swj & 31
                s.append(sj)
                w1.append(w1j)
                lim.append(w1j + 1)
                e.append(sj + w1j)
            for j in range(_U):
                v0.append(plsc.load_gather(table_v, [s[j] + lanes]))
                v1.append(plsc.load_gather(table_v, [s[j] + lanes + 16]))

            def cross(m, j):
                return ((s[m] < s[j]) & (s[j] <= e[m]) & (e[m] < e[j])) | \
                       ((s[j] < s[m]) & (s[m] <= e[j]) & (e[j] < e[m]))

            ok = []
            cs = [cnt]
            for j in range(_U):
                okj = jnp.logical_not(crosscheck(v0[j], v1[j], w1[j], lim[j]))
                for m in range(j):
                    okj = okj & jnp.logical_not(ok[m] & cross(m, j))
                okj = okj & (cs[j] < kk)
                ok.append(okj)
                cs.append(cs[j] + jnp.where(okj, jnp.int32(1), jnp.int32(0)))

            pv = jnp.zeros((16,), jnp.int32)
            maskp = jnp.zeros((16,), jnp.bool_)
            for j in range(_U):
                pv = jnp.where(lanes == 2 * j, s[j], pv)
                pv = jnp.where(lanes == 2 * j + 1, e[j], pv)
                maskp = maskp | (((lanes >> 1) == j) & ok[j])
            pre = plsc.load_gather(table_v, [pv])
            aval = jnp.int32(0)
            bval = jnp.int32(0)
            zero = jnp.zeros((16,), jnp.int32)
            for m in range(_U):
                aval = jnp.maximum(
                    aval, jnp.where(ok[m] & (pv == s[m]), lim[m], zero))
                bval = jnp.maximum(
                    bval, jnp.where(ok[m] & (pv == e[m]), lim[m], zero))
            val = (jnp.maximum(pre >> 5, aval) << 5) | \
                jnp.maximum(pre & 31, bval)
            plsc.store_scatter(table_v, [pv], val, mask=maskp)

            idxt = jnp.zeros((16,), jnp.int32)
            maskt = jnp.zeros((16,), jnp.bool_)
            for j in range(_U):
                idxt = jnp.where(lanes == j, cs[j], idxt)
                maskt = maskt | ((lanes == j) & ok[j])
            plsc.store_scatter(top_v, [idxt],
                               t0 + jnp.minimum(lanes, _U - 1),
                               mask=maskt)
            return cs[_U]

        def chunk(ci, cnt):
            return lax.cond(
                cnt < kk,
                lambda c: lax.fori_loop(ci * _CH, (ci + 1) * _CH, step, c),
                lambda c: c,
                cnt)

        lax.fori_loop(0, (_N // _U) // _CH, chunk, jnp.int32(0))
        pltpu.sync_copy(top_v, out_hbm)


def kernel(ment_starts, ment_ends, ment_scores, k):
    starts = ment_starts.astype(jnp.int32)
    ends = ment_ends.astype(jnp.int32)
    scores = jnp.asarray(ment_scores)
    order = jnp.argsort(-scores, stable=True).astype(jnp.int32)
    ssort = starts[order]
    wsort = ends[order] - ssort
    packed = ssort * 32 + wsort

    ztab = jnp.zeros((_TAB,), jnp.int32)
    fill = jnp.full((_K,), -1, jnp.int32)
    kvec = jnp.full((16,), jnp.asarray(k, jnp.int32))

    mesh = plsc.VectorSubcoreMesh(core_axis_name="c", subcore_axis_name="s",
                                  num_cores=2, num_subcores=16)
    ranks = pl.kernel(
        _greedy_body,
        out_type=jax.ShapeDtypeStruct((_K,), jnp.int32),
        mesh=mesh,
        scratch_types=[
            pltpu.VMEM((_N,), jnp.int32),
            pltpu.VMEM((_TAB,), jnp.int32),
            pltpu.VMEM((_K,), jnp.int32),
            pltpu.VMEM((16,), jnp.int32),
        ],
        compiler_params=pltpu.CompilerParams(needs_layout_passes=False),
    )(packed, ztab, fill, kvec)

    valid_sel = ranks >= 0
    top = jnp.where(valid_sel, order[jnp.where(valid_sel, ranks, 0)], -1)

    big = jnp.int32(1 << 30)
    safe_top = jnp.where(valid_sel, top, 0)
    s_sel = starts[safe_top]
    e_sel = ends[safe_top]
    m_last = ends[-1]
    a = (s_sel // 256) * m_last
    b = (s_sel % 256) * m_last + e_sel
    t_lo = (a % 256) * 256 + b
    lo = t_lo % 65536
    hi = (a // 256) + t_lo // 65536
    hi = jnp.where(valid_sel, hi, big)
    lo = jnp.where(valid_sel, lo, jnp.int32(0))
    _, _, idx = lax.sort((hi, lo, top), num_keys=2, is_stable=True)

    valid = idx >= 0
    safe = jnp.where(valid, idx, 0)
    sel_scores = jnp.where(valid, jnp.take(scores, safe), 0.0)
    return (idx, sel_scores)
